# baseline (speedup 1.0000x reference)
import functools

import jax
import jax.numpy as jnp
import numpy as np
from jax import lax
from jax.experimental import pallas as pl
from jax.experimental.pallas import tpu as pltpu
from jax.experimental.pallas import tpu_sc as plsc

_NUM_USERS = 50000
_NUM_ITEMS = 50000
_N_NODES = _NUM_USERS + _NUM_ITEMS
_D = 16
_Q = 64
_B = 4096
_NE = 3200000

_NC = 2
_NS = 16
_NW = _NC * _NS

_CH = 128
_SUP = 16
_SUPE = _SUP * _CH
_NSUP = 49
_EPT = _NSUP * _SUPE
_NE_PAD = _EPT * _NW
_TRASH = _N_NODES
_N_PAD = 100096
_ACC_ROWS = _N_PAD
_ZR = _ACC_ROWS // _NS // 16
_DUMP_ROWS = _N_PAD // _NS
_CROWS_PER_TILE = _EPT // _CH
_NBUF = 2
_CPG = 4
_NCK = _SUP // _CPG


def _vmesh():
    return plsc.VectorSubcoreMesh(core_axis_name="c", subcore_axis_name="s")


_GTR_DNUMS = lax.GatherDimensionNumbers(
    offset_dims=(), collapsed_slice_dims=(0,), start_index_map=(0,))


def _lane_bcast(v16, e2):
    idx = jnp.full((16, 1), e2, jnp.int32)
    return lax.gather(v16, idx, _GTR_DNUMS, (1,),
                      mode=lax.GatherScatterMode.PROMISE_IN_BOUNDS)


def _layer_body(p_hbm, src_hbm, dst_hbm, vals_hbm, out0, out1,
                acc, zbuf, srcv0, dstv0, valsv0,
                rows0, rows1, gsem0, gsem1, ssem0, ssem1):
    c = lax.axis_index("c")
    s = lax.axis_index("s")
    t = c * _NS + s

    rows_b = (rows0, rows1)
    gsem_b = (gsem0, gsem1)
    ssem_b = (ssem0, ssem1)

    def _zb(r, carry):
        zbuf[r] = jnp.zeros((_D,), jnp.float32)
        return carry
    lax.fori_loop(0, _ZR, _zb, 0)

    def _zc(r, carry):
        pltpu.sync_copy(zbuf, acc.at[pl.ds((s * 16 + r) * _ZR, _ZR)])
        return carry
    lax.fori_loop(0, 16, _zc, 0)
    plsc.subcore_barrier()

    _CL = _CPG * _CH

    def _gather_start(k, b, srcv):
        pltpu.make_async_copy(p_hbm.at[srcv.at[pl.ds(k * _CL, _CL)]],
                              rows_b[b], gsem_b[b]).start()

    def _gather_wait(b, srcv):
        pltpu.make_async_copy(p_hbm.at[srcv.at[pl.ds(0, _CL)]], rows_b[b],
                              gsem_b[b]).wait()

    def _scat_start(k, b, dstv):
        pltpu.make_async_copy(rows_b[b], acc.at[dstv.at[pl.ds(k * _CL, _CL)]],
                              ssem_b[b]).start(add=True)

    def _scat_wait(b, dstv):
        pltpu.make_async_copy(rows_b[b], acc.at[dstv.at[pl.ds(0, _CL)]],
                              ssem_b[b]).wait()

    def _scale(k, b, valsv):
        rb = rows_b[b]
        for q in range(_CPG):
            for g in range(_CH // 16):
                v16 = valsv[pl.ds((k * _CPG + q) * _CH + g * 16, 16)]
                for e2 in range(16):
                    e = g * 16 + e2
                    sp = _lane_bcast(v16, e2)
                    rb[e + q * _CH] = rb[e + q * _CH] * sp

    def _proc_sup(srcv, dstv, valsv):
        _gather_start(0, 0, srcv)

        def _pairck(k2, carry):
            for b in range(_NBUF):
                k = k2 * _NBUF + b
                kf = k + _NBUF - 1

                @pl.when(kf < _NCK)
                def _():
                    bf = (b - 1) % _NBUF

                    @pl.when(k >= 1)
                    def _():
                        _scat_wait(bf, dstv)
                    _gather_start(kf, bf, srcv)
                _gather_wait(b, srcv)
                _scale(k, b, valsv)
                _scat_start(k, b, dstv)
            return carry
        lax.fori_loop(0, _NCK // _NBUF, _pairck, 0)
        for b in range(_NBUF):
            _scat_wait(b, dstv)

    def _sup(j, carry):
        eoff = t * _EPT + j * _SUPE
        pltpu.sync_copy(src_hbm.at[pl.ds(eoff, _SUPE)], srcv0)
        pltpu.sync_copy(dst_hbm.at[pl.ds(eoff, _SUPE)], dstv0)
        pltpu.sync_copy(vals_hbm.at[pl.ds(eoff, _SUPE)], valsv0)
        _proc_sup(srcv0, dstv0, valsv0)
        return carry
    lax.fori_loop(0, _NSUP, _sup, 0)

    plsc.subcore_barrier()

    @pl.when(c == 0)
    def _():
        pltpu.sync_copy(acc.at[pl.ds(s * _DUMP_ROWS, _DUMP_ROWS)],
                        out0.at[pl.ds(s * _DUMP_ROWS, _DUMP_ROWS)])

    @pl.when(c == 1)
    def _():
        pltpu.sync_copy(acc.at[pl.ds(s * _DUMP_ROWS, _DUMP_ROWS)],
                        out1.at[pl.ds(s * _DUMP_ROWS, _DUMP_ROWS)])


_layer_call = functools.partial(
    pl.kernel,
    out_type=(jax.ShapeDtypeStruct((_N_PAD, _D), jnp.float32),
              jax.ShapeDtypeStruct((_N_PAD, _D), jnp.float32)),
    mesh=_vmesh(),
    compiler_params=pltpu.CompilerParams(use_tc_tiling_on_sc=False),
    scratch_types=[
        pltpu.VMEM_SHARED((_ACC_ROWS, _D), jnp.float32),
        pltpu.VMEM((_ZR, _D), jnp.float32),
        pltpu.VMEM((_SUPE,), jnp.int32),
        pltpu.VMEM((_SUPE,), jnp.int32),
        pltpu.VMEM((_SUPE,), jnp.float32),
        pltpu.VMEM((_CPG * _CH, _D), jnp.float32),
        pltpu.VMEM((_CPG * _CH, _D), jnp.float32),
        pltpu.SemaphoreType.DMA,
        pltpu.SemaphoreType.DMA,
        pltpu.SemaphoreType.DMA,
        pltpu.SemaphoreType.DMA,
    ],
)(_layer_body)


def _gather_body(p0, p1, pa, pb, bidx, uidx, aidx, widx, sidx,
                 u_t, v_t, ud_t, id_t,
                 mb_o, u_o, va_o, vw_o, vs_o, ud_o, id_o,
                 iv2, g0, g1, g2, g3, rbufq, rbufd, sem):
    c = lax.axis_index("c")
    s = lax.axis_index("s")
    t = c * _NS + s

    pltpu.sync_copy(bidx.at[pl.ds(t * 2 * _CH, 2 * _CH)], iv2)
    for k in range(2):
        ivk = iv2.at[pl.ds(k * _CH, _CH)]
        pltpu.async_copy(p0.at[ivk], g0, sem).wait()
        pltpu.async_copy(p1.at[ivk], g1, sem).wait()
        pltpu.async_copy(pa.at[ivk], g2, sem).wait()
        pltpu.async_copy(pb.at[ivk], g3, sem).wait()

        def _add(e, carry):
            g0[e] = (g0[e] + g1[e]) + (g2[e] + g3[e])
            return carry
        lax.fori_loop(0, _CH, _add, 0)
        pltpu.sync_copy(g0, mb_o.at[pl.ds(t * 2 * _CH + k * _CH, _CH)])

    def _tab(idx_hbm, table, out_ref, buf):
        pltpu.sync_copy(idx_hbm.at[pl.ds(t * _CH, _CH)], iv2.at[pl.ds(0, _CH)])
        pltpu.async_copy(table.at[iv2.at[pl.ds(0, _CH)]], buf, sem).wait()
        pltpu.sync_copy(buf, out_ref.at[pl.ds(t * _CH, _CH)])

    _tab(uidx, u_t, u_o, rbufq)
    _tab(aidx, v_t, va_o, rbufq)
    _tab(widx, v_t, vw_o, rbufq)
    _tab(sidx, v_t, vs_o, rbufq)
    _tab(uidx, ud_t, ud_o, rbufd)
    _tab(aidx, id_t, id_o, rbufd)


_gather_call = functools.partial(
    pl.kernel,
    out_type=(jax.ShapeDtypeStruct((2 * _B, _D), jnp.float32),
              jax.ShapeDtypeStruct((_B, _Q), jnp.float32),
              jax.ShapeDtypeStruct((_B, _Q), jnp.float32),
              jax.ShapeDtypeStruct((_B, _Q), jnp.float32),
              jax.ShapeDtypeStruct((_B, _Q), jnp.float32),
              jax.ShapeDtypeStruct((_B, _D), jnp.float32),
              jax.ShapeDtypeStruct((_B, _D), jnp.float32)),
    mesh=_vmesh(),
    compiler_params=pltpu.CompilerParams(use_tc_tiling_on_sc=False),
    scratch_types=[
        pltpu.VMEM((2 * _CH,), jnp.int32),
        pltpu.VMEM((_CH, _D), jnp.float32),
        pltpu.VMEM((_CH, _D), jnp.float32),
        pltpu.VMEM((_CH, _D), jnp.float32),
        pltpu.VMEM((_CH, _D), jnp.float32),
        pltpu.VMEM((_CH, _Q), jnp.float32),
        pltpu.VMEM((_CH, _D), jnp.float32),
        pltpu.SemaphoreType.DMA,
    ],
)(_gather_body)


def _combine_body(a_ref, b_ref, o_ref):
    o_ref[...] = a_ref[...] + b_ref[...]


def _combine(a, b):
    blk = _N_PAD // 16
    return pl.pallas_call(
        _combine_body,
        out_shape=jax.ShapeDtypeStruct((_N_PAD, _D), jnp.float32),
        grid=(16,),
        in_specs=[pl.BlockSpec((blk, _D), lambda i: (i, 0))] * 2,
        out_specs=pl.BlockSpec((blk, _D), lambda i: (i, 0)),
    )(a, b)


def _final_body(mb_ref, u_ref, va_ref, vw_ref, vs_ref, ud_ref, id_ref,
                um_ref, im_ref, iw_ref, o_ref):
    mu = mb_ref[0:_B, :]
    ma = mb_ref[_B:2 * _B, :]
    pref = jnp.sum(mu * ma, axis=1, keepdims=True)
    dec = jnp.sum(ud_ref[...] * id_ref[...], axis=1, keepdims=True)

    g = jnp.dot(um_ref[...], im_ref[...].T, preferred_element_type=jnp.float32)
    tmat = jnp.dot(u_ref[...], g, preferred_element_type=jnp.float32)
    s_adj = jnp.sum(tmat * va_ref[...], axis=1, keepdims=True)
    s_wk = jnp.sum(tmat * vw_ref[...], axis=1, keepdims=True)
    s_st = jnp.sum(tmat * vs_ref[...], axis=1, keepdims=True)

    iw = iw_ref[...]
    ww = jax.nn.sigmoid(jnp.log(1.0 + iw[:, 0:1]))
    sw = jax.nn.sigmoid(jnp.log(1.0 + iw[:, 1:2]))

    d_loss = jnp.sum(jnp.log(1.0 / jax.nn.sigmoid(dec))) / _B
    p_loss = jnp.sum(jnp.log(1.0 / jax.nn.sigmoid(pref))) / _B
    s_loss = jnp.sum((sw * jax.nn.softplus(s_st - s_adj)
                      + ww * jax.nn.softplus(s_wk - s_st)) * 0.5) / _B
    o_ref[...] = jnp.reshape(d_loss + p_loss + s_loss, (1, 1))


def _final(mb, u_sel, va, vw, vs, ud, idg, user_map, item_map, items_weight):
    return pl.pallas_call(
        _final_body,
        out_shape=jax.ShapeDtypeStruct((1, 1), jnp.float32),
    )(mb, u_sel, va, vw, vs, ud, idg, user_map, item_map, items_weight)


def kernel(users, adjacent_items, items_pool, items_weight, user_preference,
           item_preference, user_map, item_map, user_decision, item_decision,
           U_mul_S, V_mul_S, edge_src, edge_dst, edge_vals):
    p0 = jnp.concatenate([user_preference, item_preference], axis=0)
    pad = _NE_PAD - _NE
    src2 = jnp.concatenate([edge_src, jnp.zeros((pad,), jnp.int32)])
    dst2 = jnp.concatenate([edge_dst, jnp.full((pad,), _TRASH, jnp.int32)])
    vals2 = jnp.concatenate([edge_vals, jnp.zeros((pad,), jnp.float32)])

    a1, b1 = _layer_call(p0, src2, dst2, vals2)
    p1 = _combine(a1, b1)
    a2, b2 = _layer_call(p1, src2, dst2, vals2)

    bidx = jnp.concatenate([users, adjacent_items + _NUM_USERS])
    uidx = users
    aidx = adjacent_items
    widx = items_pool[:, 0]
    sidx = items_pool[:, 1]

    mb, u_sel, va, vw, vs, ud, idg = _gather_call(
        p0, p1, a2, b2, bidx, uidx, aidx, widx, sidx,
        U_mul_S, V_mul_S, user_decision, item_decision)

    loss = _final(mb, u_sel, va, vw, vs, ud, idg,
                  user_map, item_map, items_weight)
    return loss[0, 0]


# speedup vs baseline: 1.0525x; 1.0525x over previous
import functools

import jax
import jax.numpy as jnp
import numpy as np
from jax import lax
from jax.experimental import pallas as pl
from jax.experimental.pallas import tpu as pltpu
from jax.experimental.pallas import tpu_sc as plsc

_NUM_USERS = 50000
_NUM_ITEMS = 50000
_N_NODES = _NUM_USERS + _NUM_ITEMS
_D = 16
_Q = 64
_B = 4096
_NE = 3200000

_NC = 2
_NS = 16
_NW = _NC * _NS

_CH = 128
_SUP = 16
_SUPE = _SUP * _CH
_NSUP = 49
_EPT = _NSUP * _SUPE
_NE_PAD = _EPT * _NW
_TRASH = _N_NODES
_N_PAD = 100096
_ACC_ROWS = _N_PAD
_ZR = _ACC_ROWS // _NS // 16
_DUMP_ROWS = _N_PAD // _NS
_CROWS_PER_TILE = _EPT // _CH
_NBUF = 2
_CPG = 2
_NCK = _SUP // _CPG


def _vmesh():
    return plsc.VectorSubcoreMesh(core_axis_name="c", subcore_axis_name="s")


_GTR_DNUMS = lax.GatherDimensionNumbers(
    offset_dims=(), collapsed_slice_dims=(0,), start_index_map=(0,))


def _lane_bcast(v16, e2):
    idx = jnp.full((16, 1), e2, jnp.int32)
    return lax.gather(v16, idx, _GTR_DNUMS, (1,),
                      mode=lax.GatherScatterMode.PROMISE_IN_BOUNDS)


def _layer_body(p_hbm, src_hbm, dst_hbm, vals_hbm, out0, out1,
                acc, zbuf, srcv0, dstv0, valsv0,
                rows0, rows1, gsem0, gsem1, ssem0, ssem1):
    c = lax.axis_index("c")
    s = lax.axis_index("s")
    t = c * _NS + s

    rows_b = (rows0, rows1)
    gsem_b = (gsem0, gsem1)
    ssem_b = (ssem0, ssem1)

    def _zb(r, carry):
        zbuf[r] = jnp.zeros((_D,), jnp.float32)
        return carry
    lax.fori_loop(0, _ZR, _zb, 0)

    def _zc(r, carry):
        pltpu.sync_copy(zbuf, acc.at[pl.ds((s * 16 + r) * _ZR, _ZR)])
        return carry
    lax.fori_loop(0, 16, _zc, 0)
    plsc.subcore_barrier()

    _CL = _CPG * _CH

    def _gather_start(k, b, srcv):
        pltpu.make_async_copy(p_hbm.at[srcv.at[pl.ds(k * _CL, _CL)]],
                              rows_b[b], gsem_b[b]).start()

    def _gather_wait(b, srcv):
        pltpu.make_async_copy(p_hbm.at[srcv.at[pl.ds(0, _CL)]], rows_b[b],
                              gsem_b[b]).wait()

    def _scat_start(k, b, dstv):
        pltpu.make_async_copy(rows_b[b], acc.at[dstv.at[pl.ds(k * _CL, _CL)]],
                              ssem_b[b]).start(add=True)

    def _scat_wait(b, dstv):
        pltpu.make_async_copy(rows_b[b], acc.at[dstv.at[pl.ds(0, _CL)]],
                              ssem_b[b]).wait()

    def _scale(k, b, valsv):
        rb = rows_b[b]
        for q in range(_CPG):
            for g in range(_CH // 16):
                v16 = valsv[pl.ds((k * _CPG + q) * _CH + g * 16, 16)]
                for e2 in range(16):
                    e = g * 16 + e2
                    sp = _lane_bcast(v16, e2)
                    rb[e + q * _CH] = rb[e + q * _CH] * sp

    def _proc_sup(srcv, dstv, valsv):
        _gather_start(0, 0, srcv)

        def _pairck(k2, carry):
            for b in range(_NBUF):
                k = k2 * _NBUF + b
                kf = k + _NBUF - 1

                @pl.when(kf < _NCK)
                def _():
                    bf = (b - 1) % _NBUF

                    @pl.when(k >= 1)
                    def _():
                        _scat_wait(bf, dstv)
                    _gather_start(kf, bf, srcv)
                _gather_wait(b, srcv)
                _scale(k, b, valsv)
                _scat_start(k, b, dstv)
            return carry
        lax.fori_loop(0, _NCK // _NBUF, _pairck, 0)
        for b in range(_NBUF):
            _scat_wait(b, dstv)

    def _sup(j, carry):
        eoff = t * _EPT + j * _SUPE
        pltpu.sync_copy(src_hbm.at[pl.ds(eoff, _SUPE)], srcv0)
        pltpu.sync_copy(dst_hbm.at[pl.ds(eoff, _SUPE)], dstv0)
        pltpu.sync_copy(vals_hbm.at[pl.ds(eoff, _SUPE)], valsv0)
        _proc_sup(srcv0, dstv0, valsv0)
        return carry
    lax.fori_loop(0, _NSUP, _sup, 0)

    plsc.subcore_barrier()

    @pl.when(c == 0)
    def _():
        pltpu.sync_copy(acc.at[pl.ds(s * _DUMP_ROWS, _DUMP_ROWS)],
                        out0.at[pl.ds(s * _DUMP_ROWS, _DUMP_ROWS)])

    @pl.when(c == 1)
    def _():
        pltpu.sync_copy(acc.at[pl.ds(s * _DUMP_ROWS, _DUMP_ROWS)],
                        out1.at[pl.ds(s * _DUMP_ROWS, _DUMP_ROWS)])


_layer_call = functools.partial(
    pl.kernel,
    out_type=(jax.ShapeDtypeStruct((_N_PAD, _D), jnp.float32),
              jax.ShapeDtypeStruct((_N_PAD, _D), jnp.float32)),
    mesh=_vmesh(),
    compiler_params=pltpu.CompilerParams(use_tc_tiling_on_sc=False),
    scratch_types=[
        pltpu.VMEM_SHARED((_ACC_ROWS, _D), jnp.float32),
        pltpu.VMEM((_ZR, _D), jnp.float32),
        pltpu.VMEM((_SUPE,), jnp.int32),
        pltpu.VMEM((_SUPE,), jnp.int32),
        pltpu.VMEM((_SUPE,), jnp.float32),
        pltpu.VMEM((_CPG * _CH, _D), jnp.float32),
        pltpu.VMEM((_CPG * _CH, _D), jnp.float32),
        pltpu.SemaphoreType.DMA,
        pltpu.SemaphoreType.DMA,
        pltpu.SemaphoreType.DMA,
        pltpu.SemaphoreType.DMA,
    ],
)(_layer_body)


def _gather_body(p0, p1, pa, pb, bidx, uidx, aidx, widx, sidx,
                 u_t, v_t, ud_t, id_t,
                 mb_o, u_o, va_o, vw_o, vs_o, ud_o, id_o,
                 iv2, g0, g1, g2, g3, rbufq, rbufd, sem):
    c = lax.axis_index("c")
    s = lax.axis_index("s")
    t = c * _NS + s

    pltpu.sync_copy(bidx.at[pl.ds(t * 2 * _CH, 2 * _CH)], iv2)
    for k in range(2):
        ivk = iv2.at[pl.ds(k * _CH, _CH)]
        pltpu.async_copy(p0.at[ivk], g0, sem).wait()
        pltpu.async_copy(p1.at[ivk], g1, sem).wait()
        pltpu.async_copy(pa.at[ivk], g2, sem).wait()
        pltpu.async_copy(pb.at[ivk], g3, sem).wait()

        def _add(e, carry):
            g0[e] = (g0[e] + g1[e]) + (g2[e] + g3[e])
            return carry
        lax.fori_loop(0, _CH, _add, 0)
        pltpu.sync_copy(g0, mb_o.at[pl.ds(t * 2 * _CH + k * _CH, _CH)])

    def _tab(idx_hbm, table, out_ref, buf):
        pltpu.sync_copy(idx_hbm.at[pl.ds(t * _CH, _CH)], iv2.at[pl.ds(0, _CH)])
        pltpu.async_copy(table.at[iv2.at[pl.ds(0, _CH)]], buf, sem).wait()
        pltpu.sync_copy(buf, out_ref.at[pl.ds(t * _CH, _CH)])

    _tab(uidx, u_t, u_o, rbufq)
    _tab(aidx, v_t, va_o, rbufq)
    _tab(widx, v_t, vw_o, rbufq)
    _tab(sidx, v_t, vs_o, rbufq)
    _tab(uidx, ud_t, ud_o, rbufd)
    _tab(aidx, id_t, id_o, rbufd)


_gather_call = functools.partial(
    pl.kernel,
    out_type=(jax.ShapeDtypeStruct((2 * _B, _D), jnp.float32),
              jax.ShapeDtypeStruct((_B, _Q), jnp.float32),
              jax.ShapeDtypeStruct((_B, _Q), jnp.float32),
              jax.ShapeDtypeStruct((_B, _Q), jnp.float32),
              jax.ShapeDtypeStruct((_B, _Q), jnp.float32),
              jax.ShapeDtypeStruct((_B, _D), jnp.float32),
              jax.ShapeDtypeStruct((_B, _D), jnp.float32)),
    mesh=_vmesh(),
    compiler_params=pltpu.CompilerParams(use_tc_tiling_on_sc=False),
    scratch_types=[
        pltpu.VMEM((2 * _CH,), jnp.int32),
        pltpu.VMEM((_CH, _D), jnp.float32),
        pltpu.VMEM((_CH, _D), jnp.float32),
        pltpu.VMEM((_CH, _D), jnp.float32),
        pltpu.VMEM((_CH, _D), jnp.float32),
        pltpu.VMEM((_CH, _Q), jnp.float32),
        pltpu.VMEM((_CH, _D), jnp.float32),
        pltpu.SemaphoreType.DMA,
    ],
)(_gather_body)


def _combine_body(a_ref, b_ref, o_ref):
    o_ref[...] = a_ref[...] + b_ref[...]


def _combine(a, b):
    blk = _N_PAD // 16
    return pl.pallas_call(
        _combine_body,
        out_shape=jax.ShapeDtypeStruct((_N_PAD, _D), jnp.float32),
        grid=(16,),
        in_specs=[pl.BlockSpec((blk, _D), lambda i: (i, 0))] * 2,
        out_specs=pl.BlockSpec((blk, _D), lambda i: (i, 0)),
    )(a, b)


def _final_body(mb_ref, u_ref, va_ref, vw_ref, vs_ref, ud_ref, id_ref,
                um_ref, im_ref, iw_ref, o_ref):
    mu = mb_ref[0:_B, :]
    ma = mb_ref[_B:2 * _B, :]
    pref = jnp.sum(mu * ma, axis=1, keepdims=True)
    dec = jnp.sum(ud_ref[...] * id_ref[...], axis=1, keepdims=True)

    g = jnp.dot(um_ref[...], im_ref[...].T, preferred_element_type=jnp.float32)
    tmat = jnp.dot(u_ref[...], g, preferred_element_type=jnp.float32)
    s_adj = jnp.sum(tmat * va_ref[...], axis=1, keepdims=True)
    s_wk = jnp.sum(tmat * vw_ref[...], axis=1, keepdims=True)
    s_st = jnp.sum(tmat * vs_ref[...], axis=1, keepdims=True)

    iw = iw_ref[...]
    ww = jax.nn.sigmoid(jnp.log(1.0 + iw[:, 0:1]))
    sw = jax.nn.sigmoid(jnp.log(1.0 + iw[:, 1:2]))

    d_loss = jnp.sum(jnp.log(1.0 / jax.nn.sigmoid(dec))) / _B
    p_loss = jnp.sum(jnp.log(1.0 / jax.nn.sigmoid(pref))) / _B
    s_loss = jnp.sum((sw * jax.nn.softplus(s_st - s_adj)
                      + ww * jax.nn.softplus(s_wk - s_st)) * 0.5) / _B
    o_ref[...] = jnp.reshape(d_loss + p_loss + s_loss, (1, 1))


def _final(mb, u_sel, va, vw, vs, ud, idg, user_map, item_map, items_weight):
    return pl.pallas_call(
        _final_body,
        out_shape=jax.ShapeDtypeStruct((1, 1), jnp.float32),
    )(mb, u_sel, va, vw, vs, ud, idg, user_map, item_map, items_weight)


def kernel(users, adjacent_items, items_pool, items_weight, user_preference,
           item_preference, user_map, item_map, user_decision, item_decision,
           U_mul_S, V_mul_S, edge_src, edge_dst, edge_vals):
    p0 = jnp.concatenate([user_preference, item_preference], axis=0)
    pad = _NE_PAD - _NE
    src2 = jnp.concatenate([edge_src, jnp.zeros((pad,), jnp.int32)])
    dst2 = jnp.concatenate([edge_dst, jnp.full((pad,), _TRASH, jnp.int32)])
    vals2 = jnp.concatenate([edge_vals, jnp.zeros((pad,), jnp.float32)])

    a1, b1 = _layer_call(p0, src2, dst2, vals2)
    p1 = _combine(a1, b1)
    a2, b2 = _layer_call(p1, src2, dst2, vals2)

    bidx = jnp.concatenate([users, adjacent_items + _NUM_USERS])
    uidx = users
    aidx = adjacent_items
    widx = items_pool[:, 0]
    sidx = items_pool[:, 1]

    mb, u_sel, va, vw, vs, ud, idg = _gather_call(
        p0, p1, a2, b2, bidx, uidx, aidx, widx, sidx,
        U_mul_S, V_mul_S, user_decision, item_decision)

    loss = _final(mb, u_sel, va, vw, vs, ud, idg,
                  user_map, item_map, items_weight)
    return loss[0, 0]


# speedup vs baseline: 1.1314x; 1.0750x over previous
import functools

import jax
import jax.numpy as jnp
import numpy as np
from jax import lax
from jax.experimental import pallas as pl
from jax.experimental.pallas import tpu as pltpu
from jax.experimental.pallas import tpu_sc as plsc

_NUM_USERS = 50000
_NUM_ITEMS = 50000
_N_NODES = _NUM_USERS + _NUM_ITEMS
_D = 16
_Q = 64
_B = 4096
_NE = 3200000

_NC = 2
_NS = 16
_NW = _NC * _NS

_CH = 128
_SUP = 16
_SUPE = _SUP * _CH
_NSUP = 49
_EPT = _NSUP * _SUPE
_NSUP_LAST = 43
_TAIL_E = _NE - (_NW - 1) * _EPT - _NSUP_LAST * _SUPE
_N_PAD = 100096
_ACC_ROWS = _N_PAD
_ZR = _ACC_ROWS // _NS // 16
_DUMP_ROWS = _N_PAD // _NS
_NBUF = 2
_CPG = 2
_NCK = _SUP // _CPG
_TAIL_NCK = _TAIL_E // (_CPG * _CH)


def _vmesh():
    return plsc.VectorSubcoreMesh(core_axis_name="c", subcore_axis_name="s")


_GTR_DNUMS = lax.GatherDimensionNumbers(
    offset_dims=(), collapsed_slice_dims=(0,), start_index_map=(0,))


def _lane_bcast(v16, e2):
    idx = jnp.full((16, 1), e2, jnp.int32)
    return lax.gather(v16, idx, _GTR_DNUMS, (1,),
                      mode=lax.GatherScatterMode.PROMISE_IN_BOUNDS)


def _layer_body(p_hbm, src_hbm, dst_hbm, vals_hbm, out0, out1,
                acc, zbuf, srcv0, dstv0, valsv0,
                rows0, rows1, gsem0, gsem1, ssem0, ssem1):
    c = lax.axis_index("c")
    s = lax.axis_index("s")
    t = c * _NS + s

    rows_b = (rows0, rows1)
    gsem_b = (gsem0, gsem1)
    ssem_b = (ssem0, ssem1)

    def _zb(r, carry):
        zbuf[r] = jnp.zeros((_D,), jnp.float32)
        return carry
    lax.fori_loop(0, _ZR, _zb, 0)

    def _zc(r, carry):
        pltpu.sync_copy(zbuf, acc.at[pl.ds((s * 16 + r) * _ZR, _ZR)])
        return carry
    lax.fori_loop(0, 16, _zc, 0)
    plsc.subcore_barrier()

    _CL = _CPG * _CH

    def _gather_start(k, b, srcv):
        pltpu.make_async_copy(p_hbm.at[srcv.at[pl.ds(k * _CL, _CL)]],
                              rows_b[b], gsem_b[b]).start()

    def _gather_wait(b, srcv):
        pltpu.make_async_copy(p_hbm.at[srcv.at[pl.ds(0, _CL)]], rows_b[b],
                              gsem_b[b]).wait()

    def _scat_start(k, b, dstv):
        pltpu.make_async_copy(rows_b[b], acc.at[dstv.at[pl.ds(k * _CL, _CL)]],
                              ssem_b[b]).start(add=True)

    def _scat_wait(b, dstv):
        pltpu.make_async_copy(rows_b[b], acc.at[dstv.at[pl.ds(0, _CL)]],
                              ssem_b[b]).wait()

    def _scale(k, b, valsv):
        rb = rows_b[b]
        for q in range(_CPG):
            for g in range(_CH // 16):
                v16 = valsv[pl.ds((k * _CPG + q) * _CH + g * 16, 16)]
                for e2 in range(16):
                    e = g * 16 + e2
                    sp = _lane_bcast(v16, e2)
                    rb[e + q * _CH] = rb[e + q * _CH] * sp

    def _proc(nck, srcv, dstv, valsv):
        _gather_start(0, 0, srcv)

        def _pairck(k2, carry):
            for b in range(_NBUF):
                k = k2 * _NBUF + b
                kf = k + _NBUF - 1

                @pl.when(kf < nck)
                def _():
                    bf = (b - 1) % _NBUF

                    @pl.when(k >= 1)
                    def _():
                        _scat_wait(bf, dstv)
                    _gather_start(kf, bf, srcv)
                _gather_wait(b, srcv)
                _scale(k, b, valsv)
                _scat_start(k, b, dstv)
            return carry
        lax.fori_loop(0, nck // _NBUF, _pairck, 0)
        for b in range(_NBUF):
            _scat_wait(b, dstv)

    nsup = jnp.where(t == _NW - 1, _NSUP_LAST, _NSUP)

    def _sup(j, carry):
        eoff = t * _EPT + j * _SUPE
        pltpu.sync_copy(src_hbm.at[pl.ds(eoff, _SUPE)], srcv0)
        pltpu.sync_copy(dst_hbm.at[pl.ds(eoff, _SUPE)], dstv0)
        pltpu.sync_copy(vals_hbm.at[pl.ds(eoff, _SUPE)], valsv0)
        _proc(_NCK, srcv0, dstv0, valsv0)
        return carry
    lax.fori_loop(0, nsup, _sup, 0)

    @pl.when(t == _NW - 1)
    def _():
        eoff = t * _EPT + _NSUP_LAST * _SUPE
        pltpu.sync_copy(src_hbm.at[pl.ds(eoff, _TAIL_E)],
                        srcv0.at[pl.ds(0, _TAIL_E)])
        pltpu.sync_copy(dst_hbm.at[pl.ds(eoff, _TAIL_E)],
                        dstv0.at[pl.ds(0, _TAIL_E)])
        pltpu.sync_copy(vals_hbm.at[pl.ds(eoff, _TAIL_E)],
                        valsv0.at[pl.ds(0, _TAIL_E)])
        _proc(_TAIL_NCK, srcv0, dstv0, valsv0)

    plsc.subcore_barrier()

    @pl.when(c == 0)
    def _():
        pltpu.sync_copy(acc.at[pl.ds(s * _DUMP_ROWS, _DUMP_ROWS)],
                        out0.at[pl.ds(s * _DUMP_ROWS, _DUMP_ROWS)])

    @pl.when(c == 1)
    def _():
        pltpu.sync_copy(acc.at[pl.ds(s * _DUMP_ROWS, _DUMP_ROWS)],
                        out1.at[pl.ds(s * _DUMP_ROWS, _DUMP_ROWS)])


_layer_call = functools.partial(
    pl.kernel,
    out_type=(jax.ShapeDtypeStruct((_N_PAD, _D), jnp.float32),
              jax.ShapeDtypeStruct((_N_PAD, _D), jnp.float32)),
    mesh=_vmesh(),
    compiler_params=pltpu.CompilerParams(use_tc_tiling_on_sc=False),
    scratch_types=[
        pltpu.VMEM_SHARED((_ACC_ROWS, _D), jnp.float32),
        pltpu.VMEM((_ZR, _D), jnp.float32),
        pltpu.VMEM((_SUPE,), jnp.int32),
        pltpu.VMEM((_SUPE,), jnp.int32),
        pltpu.VMEM((_SUPE,), jnp.float32),
        pltpu.VMEM((_CPG * _CH, _D), jnp.float32),
        pltpu.VMEM((_CPG * _CH, _D), jnp.float32),
        pltpu.SemaphoreType.DMA,
        pltpu.SemaphoreType.DMA,
        pltpu.SemaphoreType.DMA,
        pltpu.SemaphoreType.DMA,
    ],
)(_layer_body)


def _gather_body(p0, p1, pa, pb, bidx, uidx, aidx, widx, sidx,
                 u_t, v_t, ud_t, id_t,
                 mb_o, u_o, va_o, vw_o, vs_o, ud_o, id_o,
                 iv2, g0, g1, g2, g3, rbufq, rbufd, sem):
    c = lax.axis_index("c")
    s = lax.axis_index("s")
    t = c * _NS + s

    pltpu.sync_copy(bidx.at[pl.ds(t * 2 * _CH, 2 * _CH)], iv2)
    for k in range(2):
        ivk = iv2.at[pl.ds(k * _CH, _CH)]
        pltpu.async_copy(p0.at[ivk], g0, sem).wait()
        pltpu.async_copy(p1.at[ivk], g1, sem).wait()
        pltpu.async_copy(pa.at[ivk], g2, sem).wait()
        pltpu.async_copy(pb.at[ivk], g3, sem).wait()

        def _add(e, carry):
            g0[e] = (g0[e] + g1[e]) + (g2[e] + g3[e])
            return carry
        lax.fori_loop(0, _CH, _add, 0)
        pltpu.sync_copy(g0, mb_o.at[pl.ds(t * 2 * _CH + k * _CH, _CH)])

    def _tab(idx_hbm, table, out_ref, buf):
        pltpu.sync_copy(idx_hbm.at[pl.ds(t * _CH, _CH)], iv2.at[pl.ds(0, _CH)])
        pltpu.async_copy(table.at[iv2.at[pl.ds(0, _CH)]], buf, sem).wait()
        pltpu.sync_copy(buf, out_ref.at[pl.ds(t * _CH, _CH)])

    _tab(uidx, u_t, u_o, rbufq)
    _tab(aidx, v_t, va_o, rbufq)
    _tab(widx, v_t, vw_o, rbufq)
    _tab(sidx, v_t, vs_o, rbufq)
    _tab(uidx, ud_t, ud_o, rbufd)
    _tab(aidx, id_t, id_o, rbufd)


_gather_call = functools.partial(
    pl.kernel,
    out_type=(jax.ShapeDtypeStruct((2 * _B, _D), jnp.float32),
              jax.ShapeDtypeStruct((_B, _Q), jnp.float32),
              jax.ShapeDtypeStruct((_B, _Q), jnp.float32),
              jax.ShapeDtypeStruct((_B, _Q), jnp.float32),
              jax.ShapeDtypeStruct((_B, _Q), jnp.float32),
              jax.ShapeDtypeStruct((_B, _D), jnp.float32),
              jax.ShapeDtypeStruct((_B, _D), jnp.float32)),
    mesh=_vmesh(),
    compiler_params=pltpu.CompilerParams(use_tc_tiling_on_sc=False),
    scratch_types=[
        pltpu.VMEM((2 * _CH,), jnp.int32),
        pltpu.VMEM((_CH, _D), jnp.float32),
        pltpu.VMEM((_CH, _D), jnp.float32),
        pltpu.VMEM((_CH, _D), jnp.float32),
        pltpu.VMEM((_CH, _D), jnp.float32),
        pltpu.VMEM((_CH, _Q), jnp.float32),
        pltpu.VMEM((_CH, _D), jnp.float32),
        pltpu.SemaphoreType.DMA,
    ],
)(_gather_body)


def _combine_body(a_ref, b_ref, o_ref):
    o_ref[...] = a_ref[...] + b_ref[...]


def _combine(a, b):
    blk = _N_PAD // 16
    return pl.pallas_call(
        _combine_body,
        out_shape=jax.ShapeDtypeStruct((_N_PAD, _D), jnp.float32),
        grid=(16,),
        in_specs=[pl.BlockSpec((blk, _D), lambda i: (i, 0))] * 2,
        out_specs=pl.BlockSpec((blk, _D), lambda i: (i, 0)),
    )(a, b)


def _final_body(mb_ref, u_ref, va_ref, vw_ref, vs_ref, ud_ref, id_ref,
                um_ref, im_ref, iw_ref, o_ref):
    mu = mb_ref[0:_B, :]
    ma = mb_ref[_B:2 * _B, :]
    pref = jnp.sum(mu * ma, axis=1, keepdims=True)
    dec = jnp.sum(ud_ref[...] * id_ref[...], axis=1, keepdims=True)

    g = jnp.dot(um_ref[...], im_ref[...].T, preferred_element_type=jnp.float32)
    tmat = jnp.dot(u_ref[...], g, preferred_element_type=jnp.float32)
    s_adj = jnp.sum(tmat * va_ref[...], axis=1, keepdims=True)
    s_wk = jnp.sum(tmat * vw_ref[...], axis=1, keepdims=True)
    s_st = jnp.sum(tmat * vs_ref[...], axis=1, keepdims=True)

    iw = iw_ref[...]
    ww = jax.nn.sigmoid(jnp.log(1.0 + iw[:, 0:1]))
    sw = jax.nn.sigmoid(jnp.log(1.0 + iw[:, 1:2]))

    d_loss = jnp.sum(jnp.log(1.0 / jax.nn.sigmoid(dec))) / _B
    p_loss = jnp.sum(jnp.log(1.0 / jax.nn.sigmoid(pref))) / _B
    s_loss = jnp.sum((sw * jax.nn.softplus(s_st - s_adj)
                      + ww * jax.nn.softplus(s_wk - s_st)) * 0.5) / _B
    o_ref[...] = jnp.reshape(d_loss + p_loss + s_loss, (1, 1))


def _final(mb, u_sel, va, vw, vs, ud, idg, user_map, item_map, items_weight):
    return pl.pallas_call(
        _final_body,
        out_shape=jax.ShapeDtypeStruct((1, 1), jnp.float32),
    )(mb, u_sel, va, vw, vs, ud, idg, user_map, item_map, items_weight)


def kernel(users, adjacent_items, items_pool, items_weight, user_preference,
           item_preference, user_map, item_map, user_decision, item_decision,
           U_mul_S, V_mul_S, edge_src, edge_dst, edge_vals):
    p0 = jnp.concatenate([user_preference, item_preference], axis=0)

    a1, b1 = _layer_call(p0, edge_src, edge_dst, edge_vals)
    p1 = _combine(a1, b1)
    a2, b2 = _layer_call(p1, edge_src, edge_dst, edge_vals)

    bidx = jnp.concatenate([users, adjacent_items + _NUM_USERS])
    uidx = users
    aidx = adjacent_items
    widx = items_pool[:, 0]
    sidx = items_pool[:, 1]

    mb, u_sel, va, vw, vs, ud, idg = _gather_call(
        p0, p1, a2, b2, bidx, uidx, aidx, widx, sidx,
        U_mul_S, V_mul_S, user_decision, item_decision)

    loss = _final(mb, u_sel, va, vw, vs, ud, idg,
                  user_map, item_map, items_weight)
    return loss[0, 0]


# speedup vs baseline: 1.2512x; 1.1059x over previous
import functools

import jax
import jax.numpy as jnp
import numpy as np
from jax import lax
from jax.experimental import pallas as pl
from jax.experimental.pallas import tpu as pltpu
from jax.experimental.pallas import tpu_sc as plsc

_NUM_USERS = 50000
_NUM_ITEMS = 50000
_N_NODES = _NUM_USERS + _NUM_ITEMS
_D = 16
_Q = 64
_B = 4096
_NE = 3200000

_NC = 2
_NS = 16
_NW = _NC * _NS

_CH = 128
_SUP = 16
_SUPE = _SUP * _CH
_NSUP = 49
_EPT = _NSUP * _SUPE
_NSUP_LAST = 43
_TAIL_E = _NE - (_NW - 1) * _EPT - _NSUP_LAST * _SUPE
_N_PAD = 100096
_ACC_ROWS = _N_PAD
_ZR = _ACC_ROWS // _NS // 16
_DUMP_ROWS = _N_PAD // _NS
_NBUF = 2
_CPG = 2
_NCK = _SUP // _CPG
_TAIL_NCK = _TAIL_E // (_CPG * _CH)


def _vmesh():
    return plsc.VectorSubcoreMesh(core_axis_name="c", subcore_axis_name="s")


_GTR_DNUMS = lax.GatherDimensionNumbers(
    offset_dims=(), collapsed_slice_dims=(0,), start_index_map=(0,))


def _lane_bcast(v16, e2):
    idx = jnp.full((16, 1), e2, jnp.int32)
    return lax.gather(v16, idx, _GTR_DNUMS, (1,),
                      mode=lax.GatherScatterMode.PROMISE_IN_BOUNDS)


def _layer_body(p_hbm, src_hbm, dst_hbm, vals_hbm, out0, out1,
                acc, zbuf, srcv0, dstv0, valsv0,
                rows0, rows1, gsem0, gsem1, ssem0, ssem1):
    c = lax.axis_index("c")
    s = lax.axis_index("s")
    t = c * _NS + s

    rows_b = (rows0, rows1)
    gsem_b = (gsem0, gsem1)
    ssem_b = (ssem0, ssem1)

    def _zb(r, carry):
        zbuf[r] = jnp.zeros((_D,), jnp.float32)
        return carry
    lax.fori_loop(0, _ZR, _zb, 0)

    def _zc(r, carry):
        pltpu.sync_copy(zbuf, acc.at[pl.ds((s * 16 + r) * _ZR, _ZR)])
        return carry
    lax.fori_loop(0, 16, _zc, 0)
    plsc.subcore_barrier()

    _CL = _CPG * _CH

    def _gather_start(k, b, srcv):
        pltpu.make_async_copy(p_hbm.at[srcv.at[pl.ds(k * _CL, _CL)]],
                              rows_b[b], gsem_b[b]).start()

    def _gather_wait(b, srcv):
        pltpu.make_async_copy(p_hbm.at[srcv.at[pl.ds(0, _CL)]], rows_b[b],
                              gsem_b[b]).wait()

    def _scat_start(k, b, dstv):
        pltpu.make_async_copy(rows_b[b], acc.at[dstv.at[pl.ds(k * _CL, _CL)]],
                              ssem_b[b]).start(add=True)

    def _scat_wait(b, dstv):
        pltpu.make_async_copy(rows_b[b], acc.at[dstv.at[pl.ds(0, _CL)]],
                              ssem_b[b]).wait()

    def _scale(k, b, valsv):
        rb = rows_b[b]
        for q in range(_CPG):
            for g in range(_CH // 16):
                v16 = valsv[pl.ds((k * _CPG + q) * _CH + g * 16, 16)]
                for e2 in range(16):
                    e = g * 16 + e2
                    sp = _lane_bcast(v16, e2)
                    rb[e + q * _CH] = rb[e + q * _CH] * sp

    def _proc(nck, srcv, dstv, valsv):
        _gather_start(0, 0, srcv)

        def _pairck(k2, carry):
            for b in range(_NBUF):
                k = k2 * _NBUF + b
                kf = k + _NBUF - 1

                @pl.when(kf < nck)
                def _():
                    bf = (b - 1) % _NBUF

                    @pl.when(k >= 1)
                    def _():
                        _scat_wait(bf, dstv)
                    _gather_start(kf, bf, srcv)
                _gather_wait(b, srcv)
                _scale(k, b, valsv)
                _scat_start(k, b, dstv)
            return carry
        lax.fori_loop(0, nck // _NBUF, _pairck, 0)
        for b in range(_NBUF):
            _scat_wait(b, dstv)

    nsup = jnp.where(t == _NW - 1, _NSUP_LAST, _NSUP)

    def _sup(j, carry):
        eoff = t * _EPT + j * _SUPE
        pltpu.sync_copy(src_hbm.at[pl.ds(eoff, _SUPE)], srcv0)
        pltpu.sync_copy(dst_hbm.at[pl.ds(eoff, _SUPE)], dstv0)
        pltpu.sync_copy(vals_hbm.at[pl.ds(eoff, _SUPE)], valsv0)
        _proc(_NCK, srcv0, dstv0, valsv0)
        return carry
    lax.fori_loop(0, nsup, _sup, 0)

    @pl.when(t == _NW - 1)
    def _():
        eoff = t * _EPT + _NSUP_LAST * _SUPE
        pltpu.sync_copy(src_hbm.at[pl.ds(eoff, _TAIL_E)],
                        srcv0.at[pl.ds(0, _TAIL_E)])
        pltpu.sync_copy(dst_hbm.at[pl.ds(eoff, _TAIL_E)],
                        dstv0.at[pl.ds(0, _TAIL_E)])
        pltpu.sync_copy(vals_hbm.at[pl.ds(eoff, _TAIL_E)],
                        valsv0.at[pl.ds(0, _TAIL_E)])
        _proc(_TAIL_NCK, srcv0, dstv0, valsv0)

    plsc.subcore_barrier()

    @pl.when(c == 0)
    def _():
        pltpu.sync_copy(acc.at[pl.ds(s * _DUMP_ROWS, _DUMP_ROWS)],
                        out0.at[pl.ds(s * _DUMP_ROWS, _DUMP_ROWS)])

    @pl.when(c == 1)
    def _():
        pltpu.sync_copy(acc.at[pl.ds(s * _DUMP_ROWS, _DUMP_ROWS)],
                        out1.at[pl.ds(s * _DUMP_ROWS, _DUMP_ROWS)])


_layer_call = functools.partial(
    pl.kernel,
    out_type=(jax.ShapeDtypeStruct((_N_PAD, _D), jnp.float32),
              jax.ShapeDtypeStruct((_N_PAD, _D), jnp.float32)),
    mesh=_vmesh(),
    compiler_params=pltpu.CompilerParams(use_tc_tiling_on_sc=False),
    scratch_types=[
        pltpu.VMEM_SHARED((_ACC_ROWS, _D), jnp.float32),
        pltpu.VMEM((_ZR, _D), jnp.float32),
        pltpu.VMEM((_SUPE,), jnp.int32),
        pltpu.VMEM((_SUPE,), jnp.int32),
        pltpu.VMEM((_SUPE,), jnp.float32),
        pltpu.VMEM((_CPG * _CH, _D), jnp.float32),
        pltpu.VMEM((_CPG * _CH, _D), jnp.float32),
        pltpu.SemaphoreType.DMA,
        pltpu.SemaphoreType.DMA,
        pltpu.SemaphoreType.DMA,
        pltpu.SemaphoreType.DMA,
    ],
)(_layer_body)


def _gather_body(p0, p1, pa, pb, bidx, uidx, aidx, widx, sidx,
                 u_t, v_t, ud_t, id_t,
                 mb_o, u_o, va_o, vw_o, vs_o, ud_o, id_o,
                 iv2, g0, g1, g2, g3, rbufq, rbufd, sem):
    c = lax.axis_index("c")
    s = lax.axis_index("s")
    t = c * _NS + s

    pltpu.sync_copy(bidx.at[pl.ds(t * 2 * _CH, 2 * _CH)], iv2)
    for k in range(2):
        ivk = iv2.at[pl.ds(k * _CH, _CH)]
        pltpu.async_copy(p0.at[ivk], g0, sem).wait()
        pltpu.async_copy(p1.at[ivk], g1, sem).wait()
        pltpu.async_copy(pa.at[ivk], g2, sem).wait()
        pltpu.async_copy(pb.at[ivk], g3, sem).wait()

        def _add(e, carry):
            g0[e] = (g0[e] + g1[e]) + (g2[e] + g3[e])
            return carry
        lax.fori_loop(0, _CH, _add, 0)
        pltpu.sync_copy(g0, mb_o.at[pl.ds(t * 2 * _CH + k * _CH, _CH)])

    def _tab(idx_hbm, table, out_ref, buf):
        pltpu.sync_copy(idx_hbm.at[pl.ds(t * _CH, _CH)], iv2.at[pl.ds(0, _CH)])
        pltpu.async_copy(table.at[iv2.at[pl.ds(0, _CH)]], buf, sem).wait()
        pltpu.sync_copy(buf, out_ref.at[pl.ds(t * _CH, _CH)])

    _tab(uidx, u_t, u_o, rbufq)
    _tab(aidx, v_t, va_o, rbufq)
    _tab(widx, v_t, vw_o, rbufq)
    _tab(sidx, v_t, vs_o, rbufq)
    _tab(uidx, ud_t, ud_o, rbufd)
    _tab(aidx, id_t, id_o, rbufd)


_gather_call = functools.partial(
    pl.kernel,
    out_type=(jax.ShapeDtypeStruct((2 * _B, _D), jnp.float32),
              jax.ShapeDtypeStruct((_B, _Q), jnp.float32),
              jax.ShapeDtypeStruct((_B, _Q), jnp.float32),
              jax.ShapeDtypeStruct((_B, _Q), jnp.float32),
              jax.ShapeDtypeStruct((_B, _Q), jnp.float32),
              jax.ShapeDtypeStruct((_B, _D), jnp.float32),
              jax.ShapeDtypeStruct((_B, _D), jnp.float32)),
    mesh=_vmesh(),
    compiler_params=pltpu.CompilerParams(use_tc_tiling_on_sc=False),
    scratch_types=[
        pltpu.VMEM((2 * _CH,), jnp.int32),
        pltpu.VMEM((_CH, _D), jnp.float32),
        pltpu.VMEM((_CH, _D), jnp.float32),
        pltpu.VMEM((_CH, _D), jnp.float32),
        pltpu.VMEM((_CH, _D), jnp.float32),
        pltpu.VMEM((_CH, _Q), jnp.float32),
        pltpu.VMEM((_CH, _D), jnp.float32),
        pltpu.SemaphoreType.DMA,
    ],
)(_gather_body)


_CB_ROWS = _N_PAD // _NW


def _combine_body(a_hbm, b_hbm, o_hbm, abuf, bbuf):
    c = lax.axis_index("c")
    s = lax.axis_index("s")
    t = c * _NS + s
    base = t * _CB_ROWS
    pltpu.sync_copy(a_hbm.at[pl.ds(base, _CB_ROWS)], abuf)
    pltpu.sync_copy(b_hbm.at[pl.ds(base, _CB_ROWS)], bbuf)

    def _ad(r8, carry):
        for u in range(8):
            r = r8 * 8 + u
            abuf[r] = abuf[r] + bbuf[r]
        return carry
    lax.fori_loop(0, _CB_ROWS // 8, _ad, 0)
    pltpu.sync_copy(abuf, o_hbm.at[pl.ds(base, _CB_ROWS)])


_combine = functools.partial(
    pl.kernel,
    out_type=jax.ShapeDtypeStruct((_N_PAD, _D), jnp.float32),
    mesh=_vmesh(),
    compiler_params=pltpu.CompilerParams(use_tc_tiling_on_sc=False),
    scratch_types=[
        pltpu.VMEM((_CB_ROWS, _D), jnp.float32),
        pltpu.VMEM((_CB_ROWS, _D), jnp.float32),
    ],
)(_combine_body)


def _final_body(mb_ref, u_ref, va_ref, vw_ref, vs_ref, ud_ref, id_ref,
                um_ref, im_ref, iw_ref, o_ref):
    mu = mb_ref[0:_B, :]
    ma = mb_ref[_B:2 * _B, :]
    pref = jnp.sum(mu * ma, axis=1, keepdims=True)
    dec = jnp.sum(ud_ref[...] * id_ref[...], axis=1, keepdims=True)

    g = jnp.dot(um_ref[...], im_ref[...].T, preferred_element_type=jnp.float32)
    tmat = jnp.dot(u_ref[...], g, preferred_element_type=jnp.float32)
    s_adj = jnp.sum(tmat * va_ref[...], axis=1, keepdims=True)
    s_wk = jnp.sum(tmat * vw_ref[...], axis=1, keepdims=True)
    s_st = jnp.sum(tmat * vs_ref[...], axis=1, keepdims=True)

    iw = iw_ref[...]
    ww = jax.nn.sigmoid(jnp.log(1.0 + iw[:, 0:1]))
    sw = jax.nn.sigmoid(jnp.log(1.0 + iw[:, 1:2]))

    d_loss = jnp.sum(jnp.log(1.0 / jax.nn.sigmoid(dec))) / _B
    p_loss = jnp.sum(jnp.log(1.0 / jax.nn.sigmoid(pref))) / _B
    s_loss = jnp.sum((sw * jax.nn.softplus(s_st - s_adj)
                      + ww * jax.nn.softplus(s_wk - s_st)) * 0.5) / _B
    o_ref[...] = jnp.reshape(d_loss + p_loss + s_loss, (1, 1))


def _final(mb, u_sel, va, vw, vs, ud, idg, user_map, item_map, items_weight):
    return pl.pallas_call(
        _final_body,
        out_shape=jax.ShapeDtypeStruct((1, 1), jnp.float32),
    )(mb, u_sel, va, vw, vs, ud, idg, user_map, item_map, items_weight)


def kernel(users, adjacent_items, items_pool, items_weight, user_preference,
           item_preference, user_map, item_map, user_decision, item_decision,
           U_mul_S, V_mul_S, edge_src, edge_dst, edge_vals):
    p0 = jnp.concatenate([user_preference, item_preference], axis=0)

    a1, b1 = _layer_call(p0, edge_src, edge_dst, edge_vals)
    p1 = _combine(a1, b1)
    a2, b2 = _layer_call(p1, edge_src, edge_dst, edge_vals)

    bidx = jnp.concatenate([users, adjacent_items + _NUM_USERS])
    uidx = users
    aidx = adjacent_items
    widx = items_pool[:, 0]
    sidx = items_pool[:, 1]

    mb, u_sel, va, vw, vs, ud, idg = _gather_call(
        p0, p1, a2, b2, bidx, uidx, aidx, widx, sidx,
        U_mul_S, V_mul_S, user_decision, item_decision)

    loss = _final(mb, u_sel, va, vw, vs, ud, idg,
                  user_map, item_map, items_weight)
    return loss[0, 0]
